# baseline (speedup 1.0000x reference)
import functools

import jax
import jax.numpy as jnp
from jax import lax
from jax.experimental import pallas as pl
from jax.experimental.pallas import tpu as pltpu
from jax.experimental.pallas import tpu_sc as plsc

B = 16384
E = 64
H = 64

_NC = 2
_NS = 16
_NW = _NC * _NS
_ROWS_PER_W = B // _NW
_CHUNK = 128
_NCHUNK = _ROWS_PER_W // _CHUNK

_TC_BLK = 1024


def _sc_gather_body(item_hbm, user_hbm, li, ri, us, p0, p1, p2,
                    gl, gr, gu, g0, g1, g2, idx_v, rows_v, sem):
    wid = lax.axis_index("s") * _NC + lax.axis_index("c")
    base = wid * _ROWS_PER_W
    jobs = [
        (li, item_hbm, gl),
        (ri, item_hbm, gr),
        (us, user_hbm, gu),
        (p0, item_hbm, g0),
        (p1, item_hbm, g1),
        (p2, item_hbm, g2),
    ]
    for idx_hbm, table, out_hbm in jobs:
        pltpu.sync_copy(idx_hbm.at[wid], idx_v)
        handles = []
        for ci in range(_NCHUNK):
            handles.append(pltpu.async_copy(
                table.at[idx_v.at[ci]],
                rows_v.at[pl.ds(ci * _CHUNK, _CHUNK)],
                sem))
        for h in handles:
            h.wait()
        pltpu.sync_copy(rows_v, out_hbm.at[pl.ds(base, _ROWS_PER_W)])


@functools.lru_cache(maxsize=1)
def _get_sc_gather():
    return pl.kernel(
        _sc_gather_body,
        out_type=[jax.ShapeDtypeStruct((B, E), jnp.float32)] * 6,
        mesh=plsc.VectorSubcoreMesh(core_axis_name="c", subcore_axis_name="s"),
        scratch_types=[
            pltpu.VMEM((_NCHUNK, _CHUNK), jnp.int32),
            pltpu.VMEM((_ROWS_PER_W, E), jnp.float32),
            pltpu.SemaphoreType.DMA,
        ],
        compiler_params=pltpu.CompilerParams(use_tc_tiling_on_sc=False),
    )


def _tc_body(gl_ref, gr_ref, gu_ref, g0_ref, g1_ref, g2_ref,
             wn_ref, bn_ref, wu_ref, bu_ref, wc_ref, bc_ref,
             w1_ref, b1_ref, wo_ref, scal_ref, out_ref):
    relu = lambda x: jnp.maximum(x, 0.0)
    mm = lambda a, w: jax.lax.dot_general(
        a, w, (((1,), (0,)), ((), ())), preferred_element_type=jnp.float32)
    d0 = scal_ref[0]
    d1 = scal_ref[1]
    d2 = scal_ref[2]
    bo = scal_ref[3]
    wn = wn_ref[...]
    bn = bn_ref[...]
    wc = wc_ref[...]
    bc = bc_ref[...]
    left = mm(relu(gl_ref[...]), wn) + bn
    right = mm(relu(gr_ref[...]), wn) + bn
    user = mm(relu(gu_ref[...]), wu_ref[...]) + bu_ref[...]
    casc = mm(relu(g2_ref[...] + d0), wc) + bc
    casc = mm(relu(casc + g1_ref[...] + d1), wc) + bc
    casc = mm(relu(casc + g0_ref[...] + d2), wc) + bc
    common = user + casc
    w1 = w1_ref[...]
    b1 = b1_ref[...]
    wo = wo_ref[...]

    def half(x):
        h = relu(mm(relu(x), w1) + b1)
        z = jnp.sum(h * wo, axis=1, keepdims=True) + bo
        return jax.nn.sigmoid(z)

    out_ref[...] = half(left + common) - half(right + common)


def _tc_forward(gl, gr, gu, g0, g1, g2, wn, bn, wu, bu, wc, bc, w1, b1, wo, scal):
    row_spec = pl.BlockSpec((_TC_BLK, E), lambda i: (i, 0))
    w_spec = pl.BlockSpec((E, H), lambda i: (0, 0))
    b_spec = pl.BlockSpec((1, H), lambda i: (0, 0))
    return pl.pallas_call(
        _tc_body,
        grid=(B // _TC_BLK,),
        in_specs=[row_spec] * 6 + [w_spec, b_spec, w_spec, b_spec, w_spec,
                                   b_spec, w_spec, b_spec, b_spec,
                                   pl.BlockSpec(memory_space=pltpu.SMEM)],
        out_specs=pl.BlockSpec((_TC_BLK, 1), lambda i: (i, 0)),
        out_shape=jax.ShapeDtypeStruct((B, 1), jnp.float32),
        compiler_params=pltpu.CompilerParams(
            dimension_semantics=("parallel",)),
    )(gl, gr, gu, g0, g1, g2, wn, bn, wu, bu, wc, bc, w1, b1, wo, scal)


def _prep_idx(x):
    return x.astype(jnp.int32).reshape(_NW, _NCHUNK, _CHUNK)


def kernel(users, left_items, right_items, prev_item_0, prev_item_1,
           prev_item_2, item_emb, user_emb, W_user, b_user, W_next, b_next,
           W_casc, b_casc, d0, d1, d2, W1, b1, Wo, bo):
    li = _prep_idx(left_items)
    ri = _prep_idx(right_items)
    us = _prep_idx(users)
    p0 = _prep_idx(prev_item_0)
    p1 = _prep_idx(prev_item_1)
    p2 = _prep_idx(prev_item_2)
    gl, gr, gu, g0, g1, g2 = _get_sc_gather()(
        item_emb, user_emb, li, ri, us, p0, p1, p2)
    scal = jnp.concatenate([d0, d1, d2, bo]).astype(jnp.float32)
    return _tc_forward(
        gl, gr, gu, g0, g1, g2,
        W_next, b_next.reshape(1, H),
        W_user, b_user.reshape(1, H),
        W_casc, b_casc.reshape(1, H),
        W1, b1.reshape(1, H),
        Wo.reshape(1, H), scal)
